# baseline (speedup 1.0000x reference)
import functools

import jax
import jax.numpy as jnp
from jax import lax
from jax.experimental import pallas as pl
from jax.experimental.pallas import tpu as pltpu
from jax.experimental.pallas import tpu_sc as plsc

_N = 10000
_E = 320000
_D = 128
_NSUB = 16
_NP = 10240
_SLICE = _NP // _NSUB
_EC = _E // _NSUB
_L = 16
_ROWS = _NP // 128
_RPT = _ROWS // _NSUB


def _rsqrt16(d):
    i = plsc.bitcast(d, jnp.int32)
    i = jnp.int32(0x5F3759DF) - lax.shift_right_logical(i, 1)
    y = plsc.bitcast(i, jnp.float32)
    for _ in range(3):
        y = y * (1.5 - 0.5 * d * y * y)
    return jnp.where(d > 0.5, y, 0.0)


def _sc_body(pk_h, u_out,
             pk_v, w_v, acc_v, acc2_v, raw_v, ns_v, nd_v, u_v, ws_v,
             zer_v, ridx_v, sh, sh2, sh_w):
    sid = lax.axis_index("s")
    ebase = sid * _EC
    nbase = sid * _SLICE
    rbase = sid * _RPT

    pltpu.sync_copy(pk_h.at[pl.ds(ebase, _EC)], pk_v)

    zeros16 = jnp.zeros((_L,), jnp.float32)
    ones16 = jnp.ones((_L,), jnp.float32)
    lomask = jnp.full((_L,), 0xFFFF, jnp.int32)
    colmask = jnp.full((_L,), 127, jnp.int32)

    for j in range(_ROWS // _L):
        ridx_v[pl.ds(j * _L, _L)] = (
            lax.broadcasted_iota(jnp.int32, (_L,), 0) + jnp.int32(j * _L))

    @plsc.parallel_loop(0, (_RPT * 128) // _L, unroll=8)
    def _(i):
        zer_v[lax.shift_right_logical(i, 3),
              pl.ds(lax.mul(lax.bitwise_and(i, 7), _L), _L)] = zeros16

    def _zero2d(ref):
        @plsc.parallel_loop(0, _NP // _L, unroll=8)
        def _(i):
            ref[lax.shift_right_logical(i, 3),
                pl.ds(lax.mul(lax.bitwise_and(i, 7), _L), _L)] = zeros16

    _zero2d(acc_v)
    _zero2d(acc2_v)
    pltpu.sync_copy(zer_v, sh.at[pl.ds(rbase, _RPT)])
    pltpu.sync_copy(zer_v, sh2.at[pl.ds(rbase, _RPT)])

    @plsc.parallel_loop(0, _EC // _L, unroll=8)
    def _(i):
        pk = pk_v[pl.ds(i * _L, _L)]
        si = lax.bitwise_and(pk, lomask)
        di = lax.shift_right_logical(pk, 16)
        plsc.addupdate_scatter(
            acc_v, [lax.shift_right_logical(si, 7),
                    lax.bitwise_and(si, colmask)], ones16)
        plsc.addupdate_scatter(
            acc2_v, [lax.shift_right_logical(di, 7),
                     lax.bitwise_and(di, colmask)], ones16)
    plsc.subcore_barrier()
    pltpu.sync_copy(acc_v, sh.at[ridx_v], add=True)
    pltpu.sync_copy(acc2_v, sh2.at[ridx_v], add=True)
    plsc.subcore_barrier()

    pltpu.sync_copy(sh.at[pl.ds(rbase, _RPT)], raw_v)

    @plsc.parallel_loop(0, _SLICE // _L, unroll=2)
    def _(j):
        d = raw_v[lax.shift_right_logical(j, 3),
                  pl.ds(lax.mul(lax.bitwise_and(j, 7), _L), _L)]
        ns_v[pl.ds(j * _L, _L)] = _rsqrt16(d)

    pltpu.sync_copy(sh2.at[pl.ds(rbase, _RPT)], raw_v)

    @plsc.parallel_loop(0, _SLICE // _L, unroll=2)
    def _(j):
        r = lax.shift_right_logical(j, 3)
        c = lax.mul(lax.bitwise_and(j, 7), _L)
        nd = _rsqrt16(raw_v[r, pl.ds(c, _L)])
        nd_v[pl.ds(j * _L, _L)] = nd
        ws_v[r, pl.ds(c, _L)] = nd

    pltpu.sync_copy(ws_v, sh_w.at[pl.ds(rbase, _RPT)])
    plsc.subcore_barrier()
    pltpu.sync_copy(sh_w, w_v)

    for k in range(5):
        _zero2d(acc_v)
        pltpu.sync_copy(zer_v, sh.at[pl.ds(rbase, _RPT)])

        @plsc.parallel_loop(0, _EC // _L, unroll=8)
        def _(i):
            pk = pk_v[pl.ds(i * _L, _L)]
            di = lax.shift_right_logical(pk, 16)
            vals = plsc.load_gather(
                w_v, [lax.shift_right_logical(di, 7),
                      lax.bitwise_and(di, colmask)])
            si = lax.bitwise_and(pk, lomask)
            plsc.addupdate_scatter(
                acc_v, [lax.shift_right_logical(si, 7),
                        lax.bitwise_and(si, colmask)], vals)
        plsc.subcore_barrier()
        pltpu.sync_copy(acc_v, sh.at[ridx_v], add=True)
        plsc.subcore_barrier()
        pltpu.sync_copy(sh.at[pl.ds(rbase, _RPT)], raw_v)

        @plsc.parallel_loop(0, _SLICE // _L, unroll=2)
        def _(j):
            r = lax.shift_right_logical(j, 3)
            c = lax.mul(lax.bitwise_and(j, 7), _L)
            u = ns_v[pl.ds(j * _L, _L)] * raw_v[r, pl.ds(c, _L)]
            u_v[pl.ds(j * _L, _L)] = u
            ws_v[r, pl.ds(c, _L)] = nd_v[pl.ds(j * _L, _L)] * u

        pltpu.sync_copy(u_v, u_out.at[pl.ds(k * _NP + nbase, _SLICE)])
        if k < 4:
            pltpu.sync_copy(ws_v, sh_w.at[pl.ds(rbase, _RPT)])
            plsc.subcore_barrier()
            pltpu.sync_copy(sh_w, w_v)


def _make_sc_prop(interpret=False):
    return pl.kernel(
        _sc_body,
        out_type=jax.ShapeDtypeStruct((5 * _NP,), jnp.float32),
        mesh=plsc.VectorSubcoreMesh(
            core_axis_name="c", subcore_axis_name="s",
            num_cores=1, num_subcores=_NSUB),
        scratch_types=[
            pltpu.VMEM((_EC,), jnp.int32),
            pltpu.VMEM((_ROWS, 128), jnp.float32),
            pltpu.VMEM((_ROWS, 128), jnp.float32),
            pltpu.VMEM((_ROWS, 128), jnp.float32),
            pltpu.VMEM((_RPT, 128), jnp.float32),
            pltpu.VMEM((_SLICE,), jnp.float32),
            pltpu.VMEM((_SLICE,), jnp.float32),
            pltpu.VMEM((_SLICE,), jnp.float32),
            pltpu.VMEM((_RPT, 128), jnp.float32),
            pltpu.VMEM((_RPT, 128), jnp.float32),
            pltpu.VMEM((_ROWS,), jnp.int32),
            pltpu.VMEM_SHARED((_ROWS, 128), jnp.float32),
            pltpu.VMEM_SHARED((_ROWS, 128), jnp.float32),
            pltpu.VMEM_SHARED((_ROWS, 128), jnp.float32),
        ],
        compiler_params=pltpu.CompilerParams(needs_layout_passes=False),
        interpret=interpret,
    )


def _tc_body(h_ref, u_ref, w1, b1, w2, b2, w3, b3, w4, b4, w5, b5,
             fcw, fcb, out_ref):
    u5 = u_ref[4:5, 0:_N]
    t = jnp.dot(u5, h_ref[...], preferred_element_type=jnp.float32)
    s1 = jnp.sum(u_ref[0, :])
    s2 = jnp.sum(u_ref[1, :])
    s3 = jnp.sum(u_ref[2, :])
    s4 = jnp.sum(u_ref[3, :])
    t = jnp.dot(t, w1[...], preferred_element_type=jnp.float32) + s4 * b1[...]
    t = jnp.dot(t, w2[...], preferred_element_type=jnp.float32) + s3 * b2[...]
    t = jnp.dot(t, w3[...], preferred_element_type=jnp.float32) + s2 * b3[...]
    t = jnp.dot(t, w4[...], preferred_element_type=jnp.float32) + s1 * b4[...]
    t = jnp.dot(t, w5[...], preferred_element_type=jnp.float32) + _N * b5[...]
    hg = t * (1.0 / _N)
    logit = jnp.sum(hg * fcw[...]) + fcb[0, 0]
    out_ref[...] = jax.nn.sigmoid(jnp.zeros((1, _D), jnp.float32) + logit)


def _make_tc_tail(interpret=False):
    return pl.pallas_call(
        _tc_body,
        out_shape=jax.ShapeDtypeStruct((1, _D), jnp.float32),
        interpret=interpret,
    )


_sc_prop = functools.cache(_make_sc_prop)
_tc_tail = functools.cache(_make_tc_tail)


def kernel(h, edge_index, W1, b1, W2, b2, W3, b3, W4, b4, W5, b5, fc_w, fc_b):
    src = edge_index[0].astype(jnp.int32)
    dst = edge_index[1].astype(jnp.int32)
    packed = jnp.bitwise_or(src, jnp.left_shift(dst, 16))
    u = _sc_prop()(packed).reshape(5, _NP)
    b1r = b1.reshape(1, _D)
    b2r = b2.reshape(1, _D)
    b3r = b3.reshape(1, _D)
    b4r = b4.reshape(1, _D)
    b5r = b5.reshape(1, _D)
    out = _tc_tail()(h, u, W1, b1r, W2, b2r, W3, b3r, W4, b4r, W5, b5r,
                     fc_w, fc_b.reshape(1, 1))
    return out[:, 0:1]


# speedup vs baseline: 1.0022x; 1.0022x over previous
import functools

import jax
import jax.numpy as jnp
from jax import lax
from jax.experimental import pallas as pl
from jax.experimental.pallas import tpu as pltpu
from jax.experimental.pallas import tpu_sc as plsc

_N = 10000
_E = 320000
_D = 128
_NSUB = 16
_NP = 10240
_SLICE = _NP // _NSUB
_EC = _E // _NSUB
_L = 16
_ROWS = _NP // 128
_RPT = _ROWS // _NSUB


def _rsqrt16(d):
    i = plsc.bitcast(d, jnp.int32)
    i = jnp.int32(0x5F3759DF) - lax.shift_right_logical(i, 1)
    y = plsc.bitcast(i, jnp.float32)
    for _ in range(3):
        y = y * (1.5 - 0.5 * d * y * y)
    return jnp.where(d > 0.5, y, 0.0)


def _sc_body(pk_h, u_out,
             pk_v, w_v, acc_v, acc2_v, raw_v, ns_v, nd_v, u_v, ws_v,
             zer_v, ridx_v, sh, sh2, sh_w, usem):
    sid = lax.axis_index("s")
    ebase = sid * _EC
    nbase = sid * _SLICE
    rbase = sid * _RPT

    pltpu.sync_copy(pk_h.at[pl.ds(ebase, _EC)], pk_v)

    zeros16 = jnp.zeros((_L,), jnp.float32)
    ones16 = jnp.ones((_L,), jnp.float32)
    lomask = jnp.full((_L,), 0xFFFF, jnp.int32)
    colmask = jnp.full((_L,), 127, jnp.int32)

    for j in range(_ROWS // _L):
        ridx_v[pl.ds(j * _L, _L)] = (
            lax.broadcasted_iota(jnp.int32, (_L,), 0) + jnp.int32(j * _L))

    @plsc.parallel_loop(0, (_RPT * 128) // _L, unroll=8)
    def _(i):
        zer_v[lax.shift_right_logical(i, 3),
              pl.ds(lax.mul(lax.bitwise_and(i, 7), _L), _L)] = zeros16

    def _zero2d(ref):
        @plsc.parallel_loop(0, _NP // _L, unroll=8)
        def _(i):
            ref[lax.shift_right_logical(i, 3),
                pl.ds(lax.mul(lax.bitwise_and(i, 7), _L), _L)] = zeros16

    _zero2d(acc_v)
    _zero2d(acc2_v)
    pltpu.sync_copy(zer_v, sh.at[pl.ds(rbase, _RPT)])
    pltpu.sync_copy(zer_v, sh2.at[pl.ds(rbase, _RPT)])

    @plsc.parallel_loop(0, _EC // _L, unroll=8)
    def _(i):
        pk = pk_v[pl.ds(i * _L, _L)]
        si = lax.bitwise_and(pk, lomask)
        di = lax.shift_right_logical(pk, 16)
        plsc.addupdate_scatter(
            acc_v, [lax.shift_right_logical(si, 7),
                    lax.bitwise_and(si, colmask)], ones16)
        plsc.addupdate_scatter(
            acc2_v, [lax.shift_right_logical(di, 7),
                     lax.bitwise_and(di, colmask)], ones16)
    plsc.subcore_barrier()
    pltpu.sync_copy(acc_v, sh.at[ridx_v], add=True)
    pltpu.sync_copy(acc2_v, sh2.at[ridx_v], add=True)
    plsc.subcore_barrier()

    pltpu.sync_copy(sh.at[pl.ds(rbase, _RPT)], raw_v)

    @plsc.parallel_loop(0, _SLICE // _L, unroll=2)
    def _(j):
        d = raw_v[lax.shift_right_logical(j, 3),
                  pl.ds(lax.mul(lax.bitwise_and(j, 7), _L), _L)]
        ns_v[pl.ds(j * _L, _L)] = _rsqrt16(d)

    pltpu.sync_copy(sh2.at[pl.ds(rbase, _RPT)], raw_v)

    @plsc.parallel_loop(0, _SLICE // _L, unroll=2)
    def _(j):
        r = lax.shift_right_logical(j, 3)
        c = lax.mul(lax.bitwise_and(j, 7), _L)
        nd = _rsqrt16(raw_v[r, pl.ds(c, _L)])
        nd_v[pl.ds(j * _L, _L)] = nd
        ws_v[r, pl.ds(c, _L)] = nd

    pltpu.sync_copy(ws_v, sh_w.at[pl.ds(rbase, _RPT)])
    plsc.subcore_barrier()
    pltpu.sync_copy(sh_w, w_v)

    u_dmas = []
    for k in range(5):
        _zero2d(acc_v)
        pltpu.sync_copy(zer_v, sh.at[pl.ds(rbase, _RPT)])

        @plsc.parallel_loop(0, _EC // _L, unroll=8)
        def _(i):
            pk = pk_v[pl.ds(i * _L, _L)]
            di = lax.shift_right_logical(pk, 16)
            vals = plsc.load_gather(
                w_v, [lax.shift_right_logical(di, 7),
                      lax.bitwise_and(di, colmask)])
            si = lax.bitwise_and(pk, lomask)
            plsc.addupdate_scatter(
                acc_v, [lax.shift_right_logical(si, 7),
                        lax.bitwise_and(si, colmask)], vals)
        plsc.subcore_barrier()
        pltpu.sync_copy(acc_v, sh.at[ridx_v], add=True)
        plsc.subcore_barrier()
        pltpu.sync_copy(sh.at[pl.ds(rbase, _RPT)], raw_v)

        @plsc.parallel_loop(0, _SLICE // _L, unroll=2)
        def _(j, k=k):
            r = lax.shift_right_logical(j, 3)
            c = lax.mul(lax.bitwise_and(j, 7), _L)
            u = ns_v[pl.ds(j * _L, _L)] * raw_v[r, pl.ds(c, _L)]
            u_v[pl.ds(k * _SLICE + j * _L, _L)] = u
            ws_v[r, pl.ds(c, _L)] = nd_v[pl.ds(j * _L, _L)] * u

        u_dmas.append(pltpu.async_copy(
            u_v.at[pl.ds(k * _SLICE, _SLICE)],
            u_out.at[pl.ds(k * _NP + nbase, _SLICE)], usem))
        if k < 4:
            pltpu.sync_copy(ws_v, sh_w.at[pl.ds(rbase, _RPT)])
            plsc.subcore_barrier()
            pltpu.sync_copy(sh_w, w_v)
    for dma in u_dmas:
        dma.wait()


def _make_sc_prop(interpret=False):
    return pl.kernel(
        _sc_body,
        out_type=jax.ShapeDtypeStruct((5 * _NP,), jnp.float32),
        mesh=plsc.VectorSubcoreMesh(
            core_axis_name="c", subcore_axis_name="s",
            num_cores=1, num_subcores=_NSUB),
        scratch_types=[
            pltpu.VMEM((_EC,), jnp.int32),
            pltpu.VMEM((_ROWS, 128), jnp.float32),
            pltpu.VMEM((_ROWS, 128), jnp.float32),
            pltpu.VMEM((_ROWS, 128), jnp.float32),
            pltpu.VMEM((_RPT, 128), jnp.float32),
            pltpu.VMEM((_SLICE,), jnp.float32),
            pltpu.VMEM((_SLICE,), jnp.float32),
            pltpu.VMEM((5 * _SLICE,), jnp.float32),
            pltpu.VMEM((_RPT, 128), jnp.float32),
            pltpu.VMEM((_RPT, 128), jnp.float32),
            pltpu.VMEM((_ROWS,), jnp.int32),
            pltpu.VMEM_SHARED((_ROWS, 128), jnp.float32),
            pltpu.VMEM_SHARED((_ROWS, 128), jnp.float32),
            pltpu.VMEM_SHARED((_ROWS, 128), jnp.float32),
            pltpu.SemaphoreType.DMA,
        ],
        compiler_params=pltpu.CompilerParams(needs_layout_passes=False),
        interpret=interpret,
    )


def _tc_body(h_ref, u_ref, w1, b1, w2, b2, w3, b3, w4, b4, w5, b5,
             fcw, fcb, out_ref):
    u5 = u_ref[4:5, 0:_N]
    t = jnp.dot(u5, h_ref[...], preferred_element_type=jnp.float32)
    s1 = jnp.sum(u_ref[0, :])
    s2 = jnp.sum(u_ref[1, :])
    s3 = jnp.sum(u_ref[2, :])
    s4 = jnp.sum(u_ref[3, :])
    t = jnp.dot(t, w1[...], preferred_element_type=jnp.float32) + s4 * b1[...]
    t = jnp.dot(t, w2[...], preferred_element_type=jnp.float32) + s3 * b2[...]
    t = jnp.dot(t, w3[...], preferred_element_type=jnp.float32) + s2 * b3[...]
    t = jnp.dot(t, w4[...], preferred_element_type=jnp.float32) + s1 * b4[...]
    t = jnp.dot(t, w5[...], preferred_element_type=jnp.float32) + _N * b5[...]
    hg = t * (1.0 / _N)
    logit = jnp.sum(hg * fcw[...]) + fcb[0, 0]
    out_ref[...] = jax.nn.sigmoid(jnp.zeros((1, _D), jnp.float32) + logit)


def _make_tc_tail(interpret=False):
    return pl.pallas_call(
        _tc_body,
        out_shape=jax.ShapeDtypeStruct((1, _D), jnp.float32),
        interpret=interpret,
    )


_sc_prop = functools.cache(_make_sc_prop)
_tc_tail = functools.cache(_make_tc_tail)


def kernel(h, edge_index, W1, b1, W2, b2, W3, b3, W4, b4, W5, b5, fc_w, fc_b):
    src = edge_index[0].astype(jnp.int32)
    dst = edge_index[1].astype(jnp.int32)
    packed = jnp.bitwise_or(src, jnp.left_shift(dst, 16))
    u = _sc_prop()(packed).reshape(5, _NP)
    b1r = b1.reshape(1, _D)
    b2r = b2.reshape(1, _D)
    b3r = b3.reshape(1, _D)
    b4r = b4.reshape(1, _D)
    b5r = b5.reshape(1, _D)
    out = _tc_tail()(h, u, W1, b1r, W2, b2r, W3, b3r, W4, b4r, W5, b5r,
                     fc_w, fc_b.reshape(1, 1))
    return out[:, 0:1]
